# Initial kernel scaffold; baseline (speedup 1.0000x reference)
import jax
import jax.numpy as jnp
from jax.experimental import pallas as pl


def kernel(x, W0, W1, b):
    raise NotImplementedError("write your pallas kernel here")


# speedup vs baseline: 124.9315x; 124.9315x over previous
import jax
import jax.numpy as jnp
from jax.experimental import pallas as pl
from jax.experimental.pallas import tpu as pltpu

L = 8
T = 2048
D = 64
DB = L * D
NUM_CELLS = L * T
NUM_TICKS = L + T - 1


def _wavefront(x_ref, w0pad_ref, b_ref, m_ref, out_ref, xw_ref, hist_ref):
    xw_ref[...] = (
        jnp.dot(x_ref[...], w0pad_ref[...], preferred_element_type=jnp.float32)
        + b_ref[...]
    )

    m = m_ref[...]

    lane_layer = jax.lax.broadcasted_iota(jnp.int32, (1, DB), 1) // D

    def warm(k, c):
        z = jnp.dot(c, m, preferred_element_type=jnp.float32) + xw_ref[pl.ds(k, 1), :]
        cn = jnp.where(k >= lane_layer, jnp.tanh(z), c)
        hist_ref[pl.ds(k, 1), :] = cn
        return cn

    def steady(k, c):
        z = (jnp.dot(c, m, preferred_element_type=jnp.float32)
             + xw_ref[pl.ds(jnp.minimum(k, T - 1), 1), :])
        cn = jnp.tanh(z)
        hist_ref[pl.ds(k, 1), :] = cn
        return cn

    c = jnp.zeros((1, DB), jnp.float32)
    c = jax.lax.fori_loop(0, L - 1, warm, c)
    c = jax.lax.fori_loop(L - 1, NUM_TICKS, steady, c)

    for l in range(L):
        out_ref[l, :, :] = hist_ref[l:l + T, l * D:(l + 1) * D]


def kernel(x, W0, W1, b):
    m = jnp.zeros((DB, DB), jnp.float32)
    for l in range(L):
        m = m.at[l * D:(l + 1) * D, l * D:(l + 1) * D].set(W1[l])
    for l in range(1, L):
        m = m.at[(l - 1) * D:l * D, l * D:(l + 1) * D].set(W0[l])
    w0pad = jnp.pad(W0[0], ((0, 0), (0, DB - D)))
    bflat = b.reshape(1, DB)

    out = pl.pallas_call(
        _wavefront,
        out_shape=jax.ShapeDtypeStruct((L, T, D), jnp.float32),
        scratch_shapes=[
            pltpu.VMEM((T, DB), jnp.float32),
            pltpu.VMEM((NUM_TICKS, DB), jnp.float32),
        ],
    )(x, w0pad, bflat, m)

    h = out.reshape(NUM_CELLS, D)
    return jnp.stack([h, h], axis=0)


# speedup vs baseline: 131.2477x; 1.0506x over previous
import jax
import jax.numpy as jnp
from jax.experimental import pallas as pl
from jax.experimental.pallas import tpu as pltpu

L = 8
T = 2048
D = 64
NUM_CELLS = L * T
NUM_TICKS = L + T - 1
NACC = 8


def _wavefront(x_ref, w00_ref, w0t_ref, w1t_ref, b_ref, out_ref, xw_ref,
               hist_ref):
    xw_ref[...] = jnp.dot(x_ref[...], w00_ref[...],
                          preferred_element_type=jnp.float32)
    bvals = b_ref[...]

    def tick_math(k, c):
        p_same = [None] * NACC
        p_up = [None] * NACC
        for d in range(D):
            bc = c[:, d:d + 1]
            t1 = bc * w1t_ref[d]
            t0 = bc * w0t_ref[d]
            j = d % NACC
            p_same[j] = t1 if p_same[j] is None else p_same[j] + t1
            p_up[j] = t0 if p_up[j] is None else p_up[j] + t0
        while len(p_same) > 1:
            p_same = [a + b for a, b in zip(p_same[0::2], p_same[1::2])]
            p_up = [a + b for a, b in zip(p_up[0::2], p_up[1::2])]
        xk = xw_ref[pl.ds(jnp.minimum(k, T - 1), 1), :]
        zup = jnp.concatenate([xk, p_up[0][:L - 1]], axis=0)
        return jnp.tanh(p_same[0] + zup + bvals)

    row = jax.lax.broadcasted_iota(jnp.int32, (L, 1), 0)

    def warm(k, c):
        cn = jnp.where(row <= k, tick_math(k, c), c)
        hist_ref[pl.ds(k, 1)] = cn[None]
        return cn

    def steady(k, c):
        cn = tick_math(k, c)
        hist_ref[pl.ds(k, 1)] = cn[None]
        return cn

    c = jnp.zeros((L, D), jnp.float32)
    c = jax.lax.fori_loop(0, L - 1, warm, c)
    c = jax.lax.fori_loop(L - 1, NUM_TICKS, steady, c)

    for l in range(L):
        out_ref[l, :, :] = hist_ref[l:l + T, l, :]


def kernel(x, W0, W1, b):
    w1t = jnp.transpose(W1, (1, 0, 2))
    w0t = jnp.pad(jnp.transpose(W0[1:], (1, 0, 2)), ((0, 0), (0, 1), (0, 0)))

    out = pl.pallas_call(
        _wavefront,
        out_shape=jax.ShapeDtypeStruct((L, T, D), jnp.float32),
        scratch_shapes=[
            pltpu.VMEM((T, D), jnp.float32),
            pltpu.VMEM((NUM_TICKS, L, D), jnp.float32),
        ],
    )(x, W0[0], w0t, w1t, b)

    h = out.reshape(NUM_CELLS, D)
    return jnp.stack([h, h], axis=0)


# speedup vs baseline: 172.3215x; 1.3129x over previous
import jax
import jax.numpy as jnp
from jax.experimental import pallas as pl
from jax.experimental.pallas import tpu as pltpu

L = 8
T = 2048
D = 64
D2 = 2 * D
NUM_CELLS = L * T
NUM_TICKS = L + T - 1


def _wavefront(xp_ref, vc0_ref, vc1_ref, bp_ref, out_ref, hist_ref):
    pltpu.matmul_push_rhs(vc0_ref[...], 0, 0)
    pltpu.matmul_push_rhs(vc1_ref[...], 0, 1)
    bp = bp_ref[...]
    row = jax.lax.broadcasted_iota(jnp.int32, (L, 1), 0)
    lane = jax.lax.broadcasted_iota(jnp.int32, (L, D2), 1)

    def piece_mask(a):
        return ((row == a) & (lane < D)) | ((row == a + 1) & (lane >= D))

    masks = [piece_mask(2 * v) for v in range(4)]

    def tick_math(k, c, latch):
        xrow = xp_ref[pl.ds(jnp.minimum(k, T - 1), 1), :]
        sh = jnp.concatenate([xrow, c[:L - 1]], axis=0)
        u = jnp.concatenate([sh, c], axis=1)
        sr = 0 if latch else None
        pltpu.matmul_acc_lhs(0, u, 0, load_staged_rhs=sr)
        pltpu.matmul_acc_lhs(0, u, 1, load_staged_rhs=sr)
        r0 = pltpu.matmul_pop(0, (L, 4 * D), jnp.float32, 0)
        r1 = pltpu.matmul_pop(0, (L, 4 * D), jnp.float32, 1)
        z = (bp
             + jnp.where(masks[0], r0[:, :D2], 0.0)
             + jnp.where(masks[1], r0[:, D2:], 0.0)
             + jnp.where(masks[2], r1[:, :D2], 0.0)
             + jnp.where(masks[3], r1[:, D2:], 0.0))
        return jnp.tanh(z)

    c = jnp.zeros((L, D2), jnp.float32)
    for k in range(L - 1):
        c = jnp.where(row <= k, tick_math(k, c, k == 0), c)
        hist_ref[pl.ds(k, 1)] = c[None]

    def steady(k, c):
        cn = tick_math(k, c, False)
        hist_ref[pl.ds(k, 1)] = cn[None]
        return cn

    c = jax.lax.fori_loop(L - 1, NUM_TICKS, steady, c)

    for l in range(L):
        off = (l % 2) * D
        out_ref[l, :, :] = hist_ref[l:l + T, l, off:off + D]


def kernel(x, W0, W1, b):
    vc = jnp.zeros((2 * D2, L * D), jnp.float32)
    for l in range(L):
        p0 = ((l - 1) % 2) * D
        p1 = D2 + (l % 2) * D
        vc = vc.at[p0:p0 + D, l * D:(l + 1) * D].set(W0[l])
        vc = vc.at[p1:p1 + D, l * D:(l + 1) * D].set(W1[l])
    xp = jnp.pad(x, ((0, 0), (D, 0)))
    bp = jnp.zeros((L, D2), jnp.float32)
    for l in range(L):
        off = (l % 2) * D
        bp = bp.at[l, off:off + D].set(b[l])

    out = pl.pallas_call(
        _wavefront,
        out_shape=jax.ShapeDtypeStruct((L, T, D), jnp.float32),
        scratch_shapes=[
            pltpu.VMEM((NUM_TICKS, L, D2), jnp.float32),
        ],
    )(xp, vc[:, :D2 * 2], vc[:, D2 * 2:], bp)

    h = out.reshape(NUM_CELLS, D)
    return jnp.stack([h, h], axis=0)
